# baseline (speedup 1.0000x reference)
import functools

import jax
import jax.numpy as jnp
from jax import lax
from jax.experimental import pallas as pl
from jax.experimental.pallas import tpu as pltpu
from jax.experimental.pallas import tpu_sc as plsc
import numpy as np

K = 4
NT = 64
B = 2500
N = 10000
E = 160000
ALPHA = 0.1
BETAS = [float(np.log(0.4 / l + 1.0)) for l in (1, 2, 3)]

F = 512
FC = 128
NCHUNK = 4
NPAD = 10240
ROWS_PER_TILE = NPAD // 16
EDGES_PER_SUB = E // 16
EB = 80
NEB = EDGES_PER_SUB // EB
EPT = E // 32

BLK = 1000
GRID = N // BLK

_SC_MESH = dict(core_axis_name="c", subcore_axis_name="s")


def _deg_sc(dst):
    mesh = plsc.VectorSubcoreMesh(**_SC_MESH)

    @functools.partial(
        pl.kernel,
        out_type=jax.ShapeDtypeStruct((2, NPAD), jnp.float32),
        mesh=mesh,
        compiler_params=pltpu.CompilerParams(needs_layout_passes=False),
        scratch_types=[
            pltpu.VMEM((NPAD,), jnp.float32),
            pltpu.VMEM((EPT + 16,), jnp.int32),
            pltpu.VMEM_SHARED((16, NPAD), jnp.float32),
            pltpu.VMEM((ROWS_PER_TILE,), jnp.float32),
            pltpu.VMEM((ROWS_PER_TILE,), jnp.float32),
        ],
    )
    def k(dst_hbm, out_hbm, hist, didx, stage, accv, tmpv):
        c = lax.axis_index("c")
        s = lax.axis_index("s")
        gid = c * 16 + s
        zeros16 = jnp.zeros((16,), jnp.float32)
        ones16 = jnp.ones((16,), jnp.float32)
        iota16 = lax.iota(jnp.int32, 16)

        def zero_body(i, _):
            hist[pl.ds(i * 16, 16)] = zeros16
            return 0
        lax.fori_loop(0, NPAD // 16, zero_body, 0)

        didx[pl.ds(EPT, 16)] = jnp.zeros((16,), jnp.int32)
        pltpu.sync_copy(dst_hbm.at[pl.ds(gid * EPT, EPT)],
                        didx.at[pl.ds(0, EPT)])

        nsteps = (EPT + 15) // 16

        def scat_body(e, _):
            off = e * 16
            idx16 = didx[pl.ds(off, 16)]
            msk = (off + iota16) < EPT
            plsc.addupdate_scatter(hist, [idx16], ones16, mask=msk)
            return 0
        lax.fori_loop(0, nsteps, scat_body, 0)

        pltpu.sync_copy(hist, stage.at[s])
        plsc.subcore_barrier()

        n0 = s * ROWS_PER_TILE
        pltpu.sync_copy(stage.at[0, pl.ds(n0, ROWS_PER_TILE)], accv)

        def comb_body(r, _):
            pltpu.sync_copy(stage.at[r, pl.ds(n0, ROWS_PER_TILE)], tmpv)

            def add_body(i, _):
                accv[pl.ds(i * 16, 16)] = (accv[pl.ds(i * 16, 16)]
                                           + tmpv[pl.ds(i * 16, 16)])
                return 0
            lax.fori_loop(0, ROWS_PER_TILE // 16, add_body, 0)
            return 0
        lax.fori_loop(1, 16, comb_body, 0)

        pltpu.sync_copy(accv, out_hbm.at[c, pl.ds(n0, ROWS_PER_TILE)])

    return k(dst)


def _scatter_sc(g0, g1, g2, g3, src1, dst2, zrows):
    mesh = plsc.VectorSubcoreMesh(**_SC_MESH)
    out_t = [jax.ShapeDtypeStruct((NPAD, FC), jnp.float32)] * 4

    @functools.partial(
        pl.kernel,
        out_type=out_t,
        mesh=mesh,
        scratch_types=[
            pltpu.VMEM_SHARED((NPAD, FC), jnp.float32),
            pltpu.VMEM((NEB * EB,), jnp.int32),
            pltpu.VMEM((NEB, EB), jnp.int32),
            pltpu.VMEM((EB, FC), jnp.float32),
            pltpu.VMEM((EB, FC), jnp.float32),
            pltpu.SemaphoreType.DMA,
            pltpu.SemaphoreType.DMA,
            pltpu.SemaphoreType.DMA,
            pltpu.SemaphoreType.DMA,
        ],
    )
    def k(g0h, g1h, g2h, g3h, srch, dsth, zh, o0, o1, o2, o3,
          acc, sidx2, didx2, rows0, rows1,
          semg0, semg1, sems0, sems1):
        c = lax.axis_index("c")
        s = lax.axis_index("s")
        t0 = s * ROWS_PER_TILE
        gs = [g0h, g1h, g2h, g3h]
        os = [o0, o1, o2, o3]
        rows = [rows0, rows1]
        semg = [semg0, semg1]
        sems = [sems0, sems1]

        pltpu.sync_copy(srch.at[pl.ds(s * NEB * EB, NEB * EB)], sidx2)
        pltpu.sync_copy(dsth.at[s], didx2)

        for chunk in range(NCHUNK):
            gk = gs[chunk]
            ok = os[chunk]

            @pl.when(c == chunk % 2)
            def _zero():
                pltpu.sync_copy(zh, acc.at[pl.ds(t0, ROWS_PER_TILE)])

            plsc.subcore_barrier()

            @pl.when(c == chunk % 2)
            def _scatter():
                pltpu.async_copy(gk.at[sidx2.at[pl.ds(0, EB)]], rows0, semg0)

                def step(b, cur, nxt):
                    pltpu.make_async_copy(
                        gk.at[sidx2.at[pl.ds(b * EB, EB)]], rows[cur],
                        semg[cur]).wait()
                    pltpu.async_copy(rows[cur], acc.at[didx2.at[b]],
                                     sems[cur], add=True)

                    @pl.when(b > 0)
                    def _():
                        pltpu.make_async_copy(
                            rows[nxt], acc.at[didx2.at[b - 1]],
                            sems[nxt]).wait()

                    @pl.when(b + 1 < NEB)
                    def _():
                        pltpu.async_copy(
                            gk.at[sidx2.at[pl.ds((b + 1) * EB, EB)]],
                            rows[nxt], semg[nxt])

                def body(i, _):
                    step(2 * i, 0, 1)
                    step(2 * i + 1, 1, 0)
                    return 0
                lax.fori_loop(0, NEB // 2, body, 0)
                if NEB % 2 == 1:
                    step(NEB - 1, 0, 1)
                last = (NEB - 1) % 2
                pltpu.make_async_copy(rows[last], acc.at[didx2.at[NEB - 1]],
                                      sems[last]).wait()

            plsc.subcore_barrier()

            @pl.when(c == chunk % 2)
            def _writeback():
                pltpu.sync_copy(acc.at[pl.ds(t0, ROWS_PER_TILE)],
                                ok.at[pl.ds(t0, ROWS_PER_TILE)])

            plsc.subcore_barrier()

    return k(g0, g1, g2, g3, src1, dst2, zrows)


def _row_spec(shape):
    nd = len(shape)
    if nd == 2:
        return pl.BlockSpec((BLK, shape[1]), lambda i: (i, 0))
    return pl.BlockSpec((shape[0], BLK, shape[2]), lambda i: (0, i, 0))


def _full_spec(shape):
    nd = len(shape)
    return pl.BlockSpec(shape, lambda i: (0,) * nd)


def _lin_tc(x, w, b, degp_t):
    def body(x_ref, w_ref, b_ref, dp_ref, h_ref, g4_ref, dis_ref):
        h = jnp.maximum(
            jnp.dot(x_ref[...], w_ref[...],
                    preferred_element_type=jnp.float32) + b_ref[...], 0.0)
        deg = dp_ref[:, 0:1] + dp_ref[:, 1:2] + 1.0
        dis = lax.rsqrt(deg)
        h_ref[...] = h
        dis_ref[...] = dis
        g = h * dis
        for c in range(NCHUNK):
            g4_ref[c] = g[:, c * FC:(c + 1) * FC]

    return pl.pallas_call(
        body,
        grid=(GRID,),
        in_specs=[
            _row_spec((N, 2 * NT)),
            _full_spec((2 * NT, F)),
            _full_spec((1, F)),
            pl.BlockSpec((BLK, 2), lambda i: (i, 0)),
        ],
        out_specs=[
            _row_spec((N, F)),
            _row_spec((NCHUNK, N, FC)),
            _row_spec((N, 1)),
        ],
        out_shape=[
            jax.ShapeDtypeStruct((N, F), jnp.float32),
            jax.ShapeDtypeStruct((NCHUNK, N, FC), jnp.float32),
            jax.ShapeDtypeStruct((N, 1), jnp.float32),
        ],
    )(x, w, b, degp_t)


def _xw_tc(h0, w2s):
    def body(h0r, w0r, w1r, w2r, o0, o1, o2):
        x0 = ALPHA * h0r[...]
        for beta, wr, o in zip(BETAS, (w0r, w1r, w2r), (o0, o1, o2)):
            o[...] = ((1.0 - beta) * x0
                      + beta * jnp.dot(x0, wr[...],
                                       preferred_element_type=jnp.float32))

    return pl.pallas_call(
        body,
        grid=(GRID,),
        in_specs=[_row_spec((N, F))] + [_full_spec((F, F))] * 3,
        out_specs=[_row_spec((N, F))] * 3,
        out_shape=[jax.ShapeDtypeStruct((N, F), jnp.float32)] * 3,
    )(h0, *w2s)


def _layer_p1(s0, s1, s2, s3, g4, xw, dis, w1, beta):
    def body(s0r, s1r, s2r, s3r, g4r, xwr, disr, w1r, out_ref, sums_ref):
        scat = jnp.concatenate([s0r[...], s1r[...], s2r[...], s3r[...]],
                               axis=1)
        g = jnp.concatenate([g4r[c] for c in range(NCHUNK)], axis=1)
        hp = (1.0 - ALPHA) * disr[...] * (scat + g)
        ob = ((1.0 - beta) * hp
              + beta * jnp.dot(hp, w1r[...],
                               preferred_element_type=jnp.float32)
              + xwr[...])
        out_ref[...] = ob

        @pl.when(pl.program_id(0) == 0)
        def _init():
            sums_ref[...] = jnp.zeros((8, F), jnp.float32)

        sums_ref[0:1, :] = sums_ref[0:1, :] + jnp.sum(ob, axis=0,
                                                      keepdims=True)
        sums_ref[1:2, :] = sums_ref[1:2, :] + jnp.sum(ob * ob, axis=0,
                                                      keepdims=True)

    return pl.pallas_call(
        body,
        grid=(GRID,),
        in_specs=[
            pl.BlockSpec((BLK, FC), lambda i: (i, 0)),
            pl.BlockSpec((BLK, FC), lambda i: (i, 0)),
            pl.BlockSpec((BLK, FC), lambda i: (i, 0)),
            pl.BlockSpec((BLK, FC), lambda i: (i, 0)),
            _row_spec((NCHUNK, N, FC)),
            _row_spec((N, F)),
            _row_spec((N, 1)),
            _full_spec((F, F)),
        ],
        out_specs=[_row_spec((N, F)), _full_spec((8, F))],
        out_shape=[jax.ShapeDtypeStruct((N, F), jnp.float32),
                   jax.ShapeDtypeStruct((8, F), jnp.float32)],
    )(s0, s1, s2, s3, g4, xw, dis, w1)


def _layer_p2(out, sums, dis, gm, bt, m, want_h):
    def body(out_ref, sums_ref, dis_ref, gm_ref, bt_ref, m_ref,
             *out_refs):
        inv_n = 1.0 / N
        mean = sums_ref[0:1, :] * inv_n
        ex2 = sums_ref[1:2, :] * inv_n
        mm = m_ref[...]
        var = ex2 + (mm * mm - 2.0 * mm) * mean * mean
        rstd = lax.rsqrt(var + 1e-5)
        ob = out_ref[...]
        h = jnp.maximum(gm_ref[...] * (ob - mm * mean) * rstd + bt_ref[...],
                        0.0)
        g4_ref = out_refs[-1]
        if want_h:
            out_refs[0][...] = h
        g = h * dis_ref[...]
        for c in range(NCHUNK):
            g4_ref[c] = g[:, c * FC:(c + 1) * FC]

    out_specs = [_row_spec((NCHUNK, N, FC))]
    out_shape = [jax.ShapeDtypeStruct((NCHUNK, N, FC), jnp.float32)]
    if want_h:
        out_specs = [_row_spec((N, F))] + out_specs
        out_shape = [jax.ShapeDtypeStruct((N, F), jnp.float32)] + out_shape

    return pl.pallas_call(
        body,
        grid=(GRID,),
        in_specs=[
            _row_spec((N, F)),
            _full_spec((8, F)),
            _row_spec((N, 1)),
            _full_spec((1, F)),
            _full_spec((1, F)),
            _full_spec((1, F)),
        ],
        out_specs=out_specs,
        out_shape=out_shape,
    )(out, sums, dis, gm, bt, m)


def _gn_fc_tc(out, sums, gm, bt, m, w, b2, din, dout):
    def body(out_ref, s_ref, gm_ref, bt_ref, m_ref, w_ref, b2_ref,
             o_ref, sums_ref):
        inv_n = 1.0 / N
        mean = s_ref[0:1, :] * inv_n
        ex2 = s_ref[1:2, :] * inv_n
        mm = m_ref[...]
        var = ex2 + (mm * mm - 2.0 * mm) * mean * mean
        rstd = lax.rsqrt(var + 1e-5)
        h = jnp.maximum(
            gm_ref[...] * (out_ref[...] - mm * mean) * rstd + bt_ref[...],
            0.0)
        o = jnp.maximum(
            jnp.dot(h, w_ref[...],
                    preferred_element_type=jnp.float32) + b2_ref[...], 0.0)
        o_ref[...] = o

        @pl.when(pl.program_id(0) == 0)
        def _init():
            sums_ref[...] = jnp.zeros((8, dout), jnp.float32)

        sums_ref[0:1, :] = sums_ref[0:1, :] + jnp.sum(o, axis=0,
                                                      keepdims=True)
        sums_ref[1:2, :] = sums_ref[1:2, :] + jnp.sum(o * o, axis=0,
                                                      keepdims=True)

    return pl.pallas_call(
        body,
        grid=(GRID,),
        in_specs=[_row_spec((N, din)), _full_spec((8, din)),
                  _full_spec((1, din)), _full_spec((1, din)),
                  _full_spec((1, din)),
                  _full_spec((din, dout)), _full_spec((1, dout))],
        out_specs=[_row_spec((N, dout)), _full_spec((8, dout))],
        out_shape=[jax.ShapeDtypeStruct((N, dout), jnp.float32),
                   jax.ShapeDtypeStruct((8, dout), jnp.float32)],
    )(out, sums, gm, bt, m, w, b2)


def _bn_fc_tc(a, sums, g, b, w, b2, din, dout):
    def body(a_ref, s_ref, g_ref, b_ref, w_ref, b2_ref, o_ref, sums_ref):
        inv_n = 1.0 / N
        mean = s_ref[0:1, :] * inv_n
        var = s_ref[1:2, :] * inv_n - mean * mean
        rstd = lax.rsqrt(var + 1e-5)
        xn = g_ref[...] * (a_ref[...] - mean) * rstd + b_ref[...]
        o = jnp.maximum(
            jnp.dot(xn, w_ref[...],
                    preferred_element_type=jnp.float32) + b2_ref[...], 0.0)
        o_ref[...] = o

        @pl.when(pl.program_id(0) == 0)
        def _init():
            sums_ref[...] = jnp.zeros((8, dout), jnp.float32)

        sums_ref[0:1, :] = sums_ref[0:1, :] + jnp.sum(o, axis=0,
                                                      keepdims=True)
        sums_ref[1:2, :] = sums_ref[1:2, :] + jnp.sum(o * o, axis=0,
                                                      keepdims=True)

    return pl.pallas_call(
        body,
        grid=(GRID,),
        in_specs=[_row_spec((N, din)), _full_spec((8, din)),
                  _full_spec((1, din)), _full_spec((1, din)),
                  _full_spec((din, dout)), _full_spec((1, dout))],
        out_specs=[_row_spec((N, dout)), _full_spec((8, dout))],
        out_shape=[jax.ShapeDtypeStruct((N, dout), jnp.float32),
                   jax.ShapeDtypeStruct((8, dout), jnp.float32)],
    )(a, sums, g, b, w, b2)


def _head_tc(a4, sums2, g2t, b2t, wrf, brf, wbb, bbb, wp, bp):
    BBLK = B
    OUTW = 2 * K * NT + 2 * K * K + K

    def body(a_ref, s_ref, g2_ref, b2_ref, wrf_ref, brf_ref,
             wbb_ref, bbb_ref, wp_ref, bp_ref, o_ref):
        inv_n = 1.0 / N
        mean = s_ref[0:1, :] * inv_n
        var = s_ref[1:2, :] * inv_n - mean * mean
        rstd = lax.rsqrt(var + 1e-5)
        mean_t = jnp.concatenate([mean] * K, axis=1)
        rstd_t = jnp.concatenate([rstd] * K, axis=1)
        hh = g2_ref[...] * (a_ref[...] - mean_t) * rstd_t + b2_ref[...]

        rf4 = jnp.dot(hh, wrf_ref[...],
                      preferred_element_type=jnp.float32) + brf_ref[...]
        bb4 = jnp.dot(hh, wbb_ref[...],
                      preferred_element_type=jnp.float32) + bbb_ref[...]
        p4 = jnp.dot(hh, wp_ref[...],
                     preferred_element_type=jnp.float32) + bp_ref[...]
        pw = 1.0 / (1.0 + jnp.exp(-p4))

        rr_l, ri_l = [], []
        for k in range(K):
            rr = rf4[:, 2 * NT * k:2 * NT * k + NT]
            ri = rf4[:, 2 * NT * k + NT:2 * NT * (k + 1)]
            den = jnp.sqrt(rr * rr + ri * ri) + 1e-12
            rr_l.append(rr / den)
            ri_l.append(ri / den)

        fro2 = jnp.zeros((BBLK, 1), jnp.float32)
        for i in range(K):
            wr = jnp.zeros((BBLK, NT), jnp.float32)
            wi = jnp.zeros((BBLK, NT), jnp.float32)
            for j in range(K):
                br = bb4[:, 2 * K * i + j:2 * K * i + j + 1]
                bi = bb4[:, 2 * K * i + K + j:2 * K * i + K + j + 1]
                wr = wr + br * rr_l[j] - bi * ri_l[j]
                wi = wi + br * ri_l[j] + bi * rr_l[j]
            fro2 = fro2 + jnp.sum(wr * wr + wi * wi, axis=1, keepdims=True)
        scale = float(np.sqrt(K)) * lax.rsqrt(fro2 + 1e-12)

        bbr_l = [bb4[:, 2 * K * i + j:2 * K * i + j + 1] * scale
                 for i in range(K) for j in range(K)]
        bbi_l = [bb4[:, 2 * K * i + K + j:2 * K * i + K + j + 1] * scale
                 for i in range(K) for j in range(K)]

        o_ref[...] = jnp.concatenate(rr_l + ri_l + bbr_l + bbi_l + [pw],
                                     axis=1)

    return pl.pallas_call(
        body,
        grid=(1,),
        in_specs=[
            pl.BlockSpec((BBLK, K * 2 * NT), lambda i: (i, 0)),
            _full_spec((8, 2 * NT)),
            _full_spec((1, K * 2 * NT)),
            _full_spec((1, K * 2 * NT)),
            _full_spec((K * 2 * NT, K * 2 * NT)),
            _full_spec((1, K * 2 * NT)),
            _full_spec((K * 2 * NT, 2 * K * K)),
            _full_spec((1, 2 * K * K)),
            _full_spec((K * 2 * NT, K)),
            _full_spec((1, K)),
        ],
        out_specs=[pl.BlockSpec((BBLK, OUTW), lambda i: (i, 0))],
        out_shape=[jax.ShapeDtypeStruct((B, OUTW), jnp.float32)],
    )(a4, sums2, g2t, b2t, wrf, brf, wbb, bbb, wp, bp)[0]


def kernel(x, params, edge_index):
    p = params
    src = edge_index[0]
    dst = edge_index[1]
    dst2 = dst.reshape(16, NEB, EB)

    degp = _deg_sc(dst)
    degp_t = jnp.transpose(degp)

    h0, g4, dis = _lin_tc(x, p['lin_w'], p['lin_b'][None, :], degp_t)

    zrows = jnp.zeros((ROWS_PER_TILE, FC), jnp.float32)
    xws = _xw_tc(h0, [p['g%d_w2' % i] for i in range(len(BETAS))])
    g4c = g4
    for i, beta in enumerate(BETAS):
        gc = [g4c[c] for c in range(NCHUNK)]
        s4 = _scatter_sc(gc[0], gc[1], gc[2], gc[3], src, dst2, zrows)
        out, sums = _layer_p1(s4[0], s4[1], s4[2], s4[3], g4c, xws[i], dis,
                              p['g%d_w1' % i], beta)
        if i == len(BETAS) - 1:
            gn_out, gn_sums = out, sums
        else:
            g4c, = _layer_p2(out, sums, dis,
                             p['n%d_g' % i][None, :], p['n%d_b' % i][None, :],
                             p['n%d_m' % i][None, :], want_h=False)

    i = len(BETAS) - 1
    a1, sums1 = _gn_fc_tc(gn_out, gn_sums,
                          p['n%d_g' % i][None, :], p['n%d_b' % i][None, :],
                          p['n%d_m' % i][None, :],
                          p['fc1_w'], p['fc1_b'][None, :], F, 256)
    a2, sums2 = _bn_fc_tc(a1, sums1, p['bn1_g'][None, :], p['bn1_b'][None, :],
                          p['fc2_w'], p['fc2_b'][None, :], 256, 128)

    a4 = a2.reshape(B, K * 2 * NT)
    eye = jnp.eye(K, dtype=jnp.float32)
    wrf = jnp.kron(eye, p['rf_w'])
    brf = jnp.tile(p['rf_b'], K)[None, :]
    wbb = jnp.kron(eye, p['bb_w'])
    bbb = jnp.tile(p['bb_b'], K)[None, :]
    wp = jnp.kron(eye, p['p_w'])
    bp = jnp.tile(p['p_b'], K)[None, :]
    g2t = jnp.tile(p['bn2_g'], K)[None, :]
    b2t = jnp.tile(p['bn2_b'], K)[None, :]

    return _head_tc(a4, sums2, g2t, b2t, wrf, brf, wbb, bbb, wp, bp)


# speedup vs baseline: 1.0059x; 1.0059x over previous
import functools

import jax
import jax.numpy as jnp
from jax import lax
from jax.experimental import pallas as pl
from jax.experimental.pallas import tpu as pltpu
from jax.experimental.pallas import tpu_sc as plsc
import numpy as np

K = 4
NT = 64
B = 2500
N = 10000
E = 160000
ALPHA = 0.1
BETAS = [float(np.log(0.4 / l + 1.0)) for l in (1, 2, 3)]

F = 512
FC = 128
NCHUNK = 4
NPAD = 10240
ROWS_PER_TILE = NPAD // 16
EDGES_PER_SUB = E // 16
EB = 80
NEB = EDGES_PER_SUB // EB
EPT = E // 32

BLK = 1000
GRID = N // BLK

_SC_MESH = dict(core_axis_name="c", subcore_axis_name="s")


def _deg_sc(dst):
    mesh = plsc.VectorSubcoreMesh(**_SC_MESH)

    @functools.partial(
        pl.kernel,
        out_type=jax.ShapeDtypeStruct((2, NPAD), jnp.float32),
        mesh=mesh,
        compiler_params=pltpu.CompilerParams(needs_layout_passes=False),
        scratch_types=[
            pltpu.VMEM((NPAD,), jnp.float32),
            pltpu.VMEM((EPT + 16,), jnp.int32),
            pltpu.VMEM_SHARED((16, NPAD), jnp.float32),
            pltpu.VMEM((ROWS_PER_TILE,), jnp.float32),
            pltpu.VMEM((ROWS_PER_TILE,), jnp.float32),
        ],
    )
    def k(dst_hbm, out_hbm, hist, didx, stage, accv, tmpv):
        c = lax.axis_index("c")
        s = lax.axis_index("s")
        gid = c * 16 + s
        zeros16 = jnp.zeros((16,), jnp.float32)
        ones16 = jnp.ones((16,), jnp.float32)
        iota16 = lax.iota(jnp.int32, 16)

        def zero_body(i, _):
            hist[pl.ds(i * 16, 16)] = zeros16
            return 0
        lax.fori_loop(0, NPAD // 16, zero_body, 0)

        didx[pl.ds(EPT, 16)] = jnp.zeros((16,), jnp.int32)
        pltpu.sync_copy(dst_hbm.at[pl.ds(gid * EPT, EPT)],
                        didx.at[pl.ds(0, EPT)])

        nsteps = (EPT + 15) // 16

        def scat_body(e, _):
            off = e * 16
            idx16 = didx[pl.ds(off, 16)]
            msk = (off + iota16) < EPT
            plsc.addupdate_scatter(hist, [idx16], ones16, mask=msk)
            return 0
        lax.fori_loop(0, nsteps, scat_body, 0)

        pltpu.sync_copy(hist, stage.at[s])
        plsc.subcore_barrier()

        n0 = s * ROWS_PER_TILE
        pltpu.sync_copy(stage.at[0, pl.ds(n0, ROWS_PER_TILE)], accv)

        def comb_body(r, _):
            pltpu.sync_copy(stage.at[r, pl.ds(n0, ROWS_PER_TILE)], tmpv)

            def add_body(i, _):
                accv[pl.ds(i * 16, 16)] = (accv[pl.ds(i * 16, 16)]
                                           + tmpv[pl.ds(i * 16, 16)])
                return 0
            lax.fori_loop(0, ROWS_PER_TILE // 16, add_body, 0)
            return 0
        lax.fori_loop(1, 16, comb_body, 0)

        pltpu.sync_copy(accv, out_hbm.at[c, pl.ds(n0, ROWS_PER_TILE)])

    return k(dst)


def _scatter_sc(g0, g1, g2, g3, src1, dst2, zrows):
    mesh = plsc.VectorSubcoreMesh(**_SC_MESH)
    out_t = [jax.ShapeDtypeStruct((NPAD, FC), jnp.float32)] * 4

    @functools.partial(
        pl.kernel,
        out_type=out_t,
        mesh=mesh,
        scratch_types=[
            pltpu.VMEM_SHARED((NPAD, FC), jnp.float32),
            pltpu.VMEM((NEB * EB,), jnp.int32),
            pltpu.VMEM((NEB, EB), jnp.int32),
            pltpu.VMEM((EB, FC), jnp.float32),
            pltpu.VMEM((EB, FC), jnp.float32),
            pltpu.SemaphoreType.DMA,
            pltpu.SemaphoreType.DMA,
            pltpu.SemaphoreType.DMA,
            pltpu.SemaphoreType.DMA,
        ],
    )
    def k(g0h, g1h, g2h, g3h, srch, dsth, zh, o0, o1, o2, o3,
          acc, sidx2, didx2, rows0, rows1,
          semg0, semg1, sems0, sems1):
        c = lax.axis_index("c")
        s = lax.axis_index("s")
        t0 = s * ROWS_PER_TILE
        gs = [g0h, g1h, g2h, g3h]
        os = [o0, o1, o2, o3]
        rows = [rows0, rows1]
        semg = [semg0, semg1]
        sems = [sems0, sems1]

        pltpu.sync_copy(srch.at[pl.ds(s * NEB * EB, NEB * EB)], sidx2)
        pltpu.sync_copy(dsth.at[s], didx2)

        for chunk in range(NCHUNK):
            gk = gs[chunk]
            ok = os[chunk]

            @pl.when(c == chunk % 2)
            def _zero():
                pltpu.sync_copy(zh, acc.at[pl.ds(t0, ROWS_PER_TILE)])

            plsc.subcore_barrier()

            @pl.when(c == chunk % 2)
            def _scatter():
                pltpu.async_copy(gk.at[sidx2.at[pl.ds(0, EB)]], rows0, semg0)

                def step(b, cur, nxt):
                    pltpu.make_async_copy(
                        gk.at[sidx2.at[pl.ds(b * EB, EB)]], rows[cur],
                        semg[cur]).wait()
                    pltpu.async_copy(rows[cur], acc.at[didx2.at[b]],
                                     sems[cur], add=True)

                    @pl.when(b > 0)
                    def _():
                        pltpu.make_async_copy(
                            rows[nxt], acc.at[didx2.at[b - 1]],
                            sems[nxt]).wait()

                    @pl.when(b + 1 < NEB)
                    def _():
                        pltpu.async_copy(
                            gk.at[sidx2.at[pl.ds((b + 1) * EB, EB)]],
                            rows[nxt], semg[nxt])

                def body(i, _):
                    step(2 * i, 0, 1)
                    step(2 * i + 1, 1, 0)
                    return 0
                lax.fori_loop(0, NEB // 2, body, 0)
                if NEB % 2 == 1:
                    step(NEB - 1, 0, 1)
                last = (NEB - 1) % 2
                pltpu.make_async_copy(rows[last], acc.at[didx2.at[NEB - 1]],
                                      sems[last]).wait()

            plsc.subcore_barrier()

            @pl.when(c == chunk % 2)
            def _writeback():
                pltpu.sync_copy(acc.at[pl.ds(t0, ROWS_PER_TILE)],
                                ok.at[pl.ds(t0, ROWS_PER_TILE)])

            plsc.subcore_barrier()

    return k(g0, g1, g2, g3, src1, dst2, zrows)


def _row_spec(shape):
    nd = len(shape)
    if nd == 2:
        return pl.BlockSpec((BLK, shape[1]), lambda i: (i, 0))
    return pl.BlockSpec((shape[0], BLK, shape[2]), lambda i: (0, i, 0))


def _full_spec(shape):
    nd = len(shape)
    return pl.BlockSpec(shape, lambda i: (0,) * nd)


def _lin_tc(x, w, b, degp_t):
    def body(x_ref, w_ref, b_ref, dp_ref, h_ref, g4_ref, dis_ref):
        h = jnp.maximum(
            jnp.dot(x_ref[...], w_ref[...],
                    preferred_element_type=jnp.float32) + b_ref[...], 0.0)
        deg = dp_ref[:, 0:1] + dp_ref[:, 1:2] + 1.0
        dis = lax.rsqrt(deg)
        h_ref[...] = h
        dis_ref[...] = dis
        g = h * dis
        for c in range(NCHUNK):
            g4_ref[c] = g[:, c * FC:(c + 1) * FC]

    return pl.pallas_call(
        body,
        grid=(GRID,),
        in_specs=[
            _row_spec((N, 2 * NT)),
            _full_spec((2 * NT, F)),
            _full_spec((1, F)),
            pl.BlockSpec((BLK, 2), lambda i: (i, 0)),
        ],
        out_specs=[
            _row_spec((N, F)),
            _row_spec((NCHUNK, N, FC)),
            _row_spec((N, 1)),
        ],
        out_shape=[
            jax.ShapeDtypeStruct((N, F), jnp.float32),
            jax.ShapeDtypeStruct((NCHUNK, N, FC), jnp.float32),
            jax.ShapeDtypeStruct((N, 1), jnp.float32),
        ],
    )(x, w, b, degp_t)


def _layer_p1(s0, s1, s2, s3, g4, h0, dis, w1, w2, beta):
    def body(s0r, s1r, s2r, s3r, g4r, h0r, disr, w1r, w2r, out_ref, sums_ref):
        scat = jnp.concatenate([s0r[...], s1r[...], s2r[...], s3r[...]],
                               axis=1)
        g = jnp.concatenate([g4r[c] for c in range(NCHUNK)], axis=1)
        hp = (1.0 - ALPHA) * disr[...] * (scat + g)
        x0 = ALPHA * h0r[...]
        ob = ((1.0 - beta) * hp
              + beta * jnp.dot(hp, w1r[...],
                               preferred_element_type=jnp.float32)
              + (1.0 - beta) * x0
              + beta * jnp.dot(x0, w2r[...],
                               preferred_element_type=jnp.float32))
        out_ref[...] = ob

        @pl.when(pl.program_id(0) == 0)
        def _init():
            sums_ref[...] = jnp.zeros((8, F), jnp.float32)

        sums_ref[0:1, :] = sums_ref[0:1, :] + jnp.sum(ob, axis=0,
                                                      keepdims=True)
        sums_ref[1:2, :] = sums_ref[1:2, :] + jnp.sum(ob * ob, axis=0,
                                                      keepdims=True)

    return pl.pallas_call(
        body,
        grid=(GRID,),
        in_specs=[
            pl.BlockSpec((BLK, FC), lambda i: (i, 0)),
            pl.BlockSpec((BLK, FC), lambda i: (i, 0)),
            pl.BlockSpec((BLK, FC), lambda i: (i, 0)),
            pl.BlockSpec((BLK, FC), lambda i: (i, 0)),
            _row_spec((NCHUNK, N, FC)),
            _row_spec((N, F)),
            _row_spec((N, 1)),
            _full_spec((F, F)),
            _full_spec((F, F)),
        ],
        out_specs=[_row_spec((N, F)), _full_spec((8, F))],
        out_shape=[jax.ShapeDtypeStruct((N, F), jnp.float32),
                   jax.ShapeDtypeStruct((8, F), jnp.float32)],
    )(s0, s1, s2, s3, g4, h0, dis, w1, w2)


def _layer_p2(out, sums, dis, gm, bt, m, want_h):
    def body(out_ref, sums_ref, dis_ref, gm_ref, bt_ref, m_ref,
             *out_refs):
        inv_n = 1.0 / N
        mean = sums_ref[0:1, :] * inv_n
        ex2 = sums_ref[1:2, :] * inv_n
        mm = m_ref[...]
        var = ex2 + (mm * mm - 2.0 * mm) * mean * mean
        rstd = lax.rsqrt(var + 1e-5)
        ob = out_ref[...]
        h = jnp.maximum(gm_ref[...] * (ob - mm * mean) * rstd + bt_ref[...],
                        0.0)
        g4_ref = out_refs[-1]
        if want_h:
            out_refs[0][...] = h
        g = h * dis_ref[...]
        for c in range(NCHUNK):
            g4_ref[c] = g[:, c * FC:(c + 1) * FC]

    out_specs = [_row_spec((NCHUNK, N, FC))]
    out_shape = [jax.ShapeDtypeStruct((NCHUNK, N, FC), jnp.float32)]
    if want_h:
        out_specs = [_row_spec((N, F))] + out_specs
        out_shape = [jax.ShapeDtypeStruct((N, F), jnp.float32)] + out_shape

    return pl.pallas_call(
        body,
        grid=(GRID,),
        in_specs=[
            _row_spec((N, F)),
            _full_spec((8, F)),
            _row_spec((N, 1)),
            _full_spec((1, F)),
            _full_spec((1, F)),
            _full_spec((1, F)),
        ],
        out_specs=out_specs,
        out_shape=out_shape,
    )(out, sums, dis, gm, bt, m)


def _gn_fc_tc(out, sums, gm, bt, m, w, b2, din, dout):
    def body(out_ref, s_ref, gm_ref, bt_ref, m_ref, w_ref, b2_ref,
             o_ref, sums_ref):
        inv_n = 1.0 / N
        mean = s_ref[0:1, :] * inv_n
        ex2 = s_ref[1:2, :] * inv_n
        mm = m_ref[...]
        var = ex2 + (mm * mm - 2.0 * mm) * mean * mean
        rstd = lax.rsqrt(var + 1e-5)
        h = jnp.maximum(
            gm_ref[...] * (out_ref[...] - mm * mean) * rstd + bt_ref[...],
            0.0)
        o = jnp.maximum(
            jnp.dot(h, w_ref[...],
                    preferred_element_type=jnp.float32) + b2_ref[...], 0.0)
        o_ref[...] = o

        @pl.when(pl.program_id(0) == 0)
        def _init():
            sums_ref[...] = jnp.zeros((8, dout), jnp.float32)

        sums_ref[0:1, :] = sums_ref[0:1, :] + jnp.sum(o, axis=0,
                                                      keepdims=True)
        sums_ref[1:2, :] = sums_ref[1:2, :] + jnp.sum(o * o, axis=0,
                                                      keepdims=True)

    return pl.pallas_call(
        body,
        grid=(GRID,),
        in_specs=[_row_spec((N, din)), _full_spec((8, din)),
                  _full_spec((1, din)), _full_spec((1, din)),
                  _full_spec((1, din)),
                  _full_spec((din, dout)), _full_spec((1, dout))],
        out_specs=[_row_spec((N, dout)), _full_spec((8, dout))],
        out_shape=[jax.ShapeDtypeStruct((N, dout), jnp.float32),
                   jax.ShapeDtypeStruct((8, dout), jnp.float32)],
    )(out, sums, gm, bt, m, w, b2)


def _bn_fc_tc(a, sums, g, b, w, b2, din, dout):
    def body(a_ref, s_ref, g_ref, b_ref, w_ref, b2_ref, o_ref, sums_ref):
        inv_n = 1.0 / N
        mean = s_ref[0:1, :] * inv_n
        var = s_ref[1:2, :] * inv_n - mean * mean
        rstd = lax.rsqrt(var + 1e-5)
        xn = g_ref[...] * (a_ref[...] - mean) * rstd + b_ref[...]
        o = jnp.maximum(
            jnp.dot(xn, w_ref[...],
                    preferred_element_type=jnp.float32) + b2_ref[...], 0.0)
        o_ref[...] = o

        @pl.when(pl.program_id(0) == 0)
        def _init():
            sums_ref[...] = jnp.zeros((8, dout), jnp.float32)

        sums_ref[0:1, :] = sums_ref[0:1, :] + jnp.sum(o, axis=0,
                                                      keepdims=True)
        sums_ref[1:2, :] = sums_ref[1:2, :] + jnp.sum(o * o, axis=0,
                                                      keepdims=True)

    return pl.pallas_call(
        body,
        grid=(GRID,),
        in_specs=[_row_spec((N, din)), _full_spec((8, din)),
                  _full_spec((1, din)), _full_spec((1, din)),
                  _full_spec((din, dout)), _full_spec((1, dout))],
        out_specs=[_row_spec((N, dout)), _full_spec((8, dout))],
        out_shape=[jax.ShapeDtypeStruct((N, dout), jnp.float32),
                   jax.ShapeDtypeStruct((8, dout), jnp.float32)],
    )(a, sums, g, b, w, b2)


def _head_tc(a4, sums2, g2t, b2t, wrf, brf, wbb, bbb, wp, bp):
    BBLK = B
    OUTW = 2 * K * NT + 2 * K * K + K

    def body(a_ref, s_ref, g2_ref, b2_ref, wrf_ref, brf_ref,
             wbb_ref, bbb_ref, wp_ref, bp_ref, o_ref):
        inv_n = 1.0 / N
        mean = s_ref[0:1, :] * inv_n
        var = s_ref[1:2, :] * inv_n - mean * mean
        rstd = lax.rsqrt(var + 1e-5)
        mean_t = jnp.concatenate([mean] * K, axis=1)
        rstd_t = jnp.concatenate([rstd] * K, axis=1)
        hh = g2_ref[...] * (a_ref[...] - mean_t) * rstd_t + b2_ref[...]

        rf4 = jnp.dot(hh, wrf_ref[...],
                      preferred_element_type=jnp.float32) + brf_ref[...]
        bb4 = jnp.dot(hh, wbb_ref[...],
                      preferred_element_type=jnp.float32) + bbb_ref[...]
        p4 = jnp.dot(hh, wp_ref[...],
                     preferred_element_type=jnp.float32) + bp_ref[...]
        pw = 1.0 / (1.0 + jnp.exp(-p4))

        rr_l, ri_l = [], []
        for k in range(K):
            rr = rf4[:, 2 * NT * k:2 * NT * k + NT]
            ri = rf4[:, 2 * NT * k + NT:2 * NT * (k + 1)]
            den = jnp.sqrt(rr * rr + ri * ri) + 1e-12
            rr_l.append(rr / den)
            ri_l.append(ri / den)

        fro2 = jnp.zeros((BBLK, 1), jnp.float32)
        for i in range(K):
            wr = jnp.zeros((BBLK, NT), jnp.float32)
            wi = jnp.zeros((BBLK, NT), jnp.float32)
            for j in range(K):
                br = bb4[:, 2 * K * i + j:2 * K * i + j + 1]
                bi = bb4[:, 2 * K * i + K + j:2 * K * i + K + j + 1]
                wr = wr + br * rr_l[j] - bi * ri_l[j]
                wi = wi + br * ri_l[j] + bi * rr_l[j]
            fro2 = fro2 + jnp.sum(wr * wr + wi * wi, axis=1, keepdims=True)
        scale = float(np.sqrt(K)) * lax.rsqrt(fro2 + 1e-12)

        bbr_l = [bb4[:, 2 * K * i + j:2 * K * i + j + 1] * scale
                 for i in range(K) for j in range(K)]
        bbi_l = [bb4[:, 2 * K * i + K + j:2 * K * i + K + j + 1] * scale
                 for i in range(K) for j in range(K)]

        o_ref[...] = jnp.concatenate(rr_l + ri_l + bbr_l + bbi_l + [pw],
                                     axis=1)

    return pl.pallas_call(
        body,
        grid=(1,),
        in_specs=[
            pl.BlockSpec((BBLK, K * 2 * NT), lambda i: (i, 0)),
            _full_spec((8, 2 * NT)),
            _full_spec((1, K * 2 * NT)),
            _full_spec((1, K * 2 * NT)),
            _full_spec((K * 2 * NT, K * 2 * NT)),
            _full_spec((1, K * 2 * NT)),
            _full_spec((K * 2 * NT, 2 * K * K)),
            _full_spec((1, 2 * K * K)),
            _full_spec((K * 2 * NT, K)),
            _full_spec((1, K)),
        ],
        out_specs=[pl.BlockSpec((BBLK, OUTW), lambda i: (i, 0))],
        out_shape=[jax.ShapeDtypeStruct((B, OUTW), jnp.float32)],
    )(a4, sums2, g2t, b2t, wrf, brf, wbb, bbb, wp, bp)[0]


def kernel(x, params, edge_index):
    p = params
    src = edge_index[0]
    dst = edge_index[1]
    dst2 = dst.reshape(16, NEB, EB)

    degp = _deg_sc(dst)
    degp_t = jnp.transpose(degp)

    h0, g4, dis = _lin_tc(x, p['lin_w'], p['lin_b'][None, :], degp_t)

    zrows = jnp.zeros((ROWS_PER_TILE, FC), jnp.float32)
    g4c = g4
    for i, beta in enumerate(BETAS):
        gc = [g4c[c] for c in range(NCHUNK)]
        s4 = _scatter_sc(gc[0], gc[1], gc[2], gc[3], src, dst2, zrows)
        out, sums = _layer_p1(s4[0], s4[1], s4[2], s4[3], g4c, h0, dis,
                              p['g%d_w1' % i], p['g%d_w2' % i], beta)
        if i == len(BETAS) - 1:
            gn_out, gn_sums = out, sums
        else:
            g4c, = _layer_p2(out, sums, dis,
                             p['n%d_g' % i][None, :], p['n%d_b' % i][None, :],
                             p['n%d_m' % i][None, :], want_h=False)

    i = len(BETAS) - 1
    a1, sums1 = _gn_fc_tc(gn_out, gn_sums,
                          p['n%d_g' % i][None, :], p['n%d_b' % i][None, :],
                          p['n%d_m' % i][None, :],
                          p['fc1_w'], p['fc1_b'][None, :], F, 256)
    a2, sums2 = _bn_fc_tc(a1, sums1, p['bn1_g'][None, :], p['bn1_b'][None, :],
                          p['fc2_w'], p['fc2_b'][None, :], 256, 128)

    a4 = a2.reshape(B, K * 2 * NT)
    eye = jnp.eye(K, dtype=jnp.float32)
    wrf = jnp.kron(eye, p['rf_w'])
    brf = jnp.tile(p['rf_b'], K)[None, :]
    wbb = jnp.kron(eye, p['bb_w'])
    bbb = jnp.tile(p['bb_b'], K)[None, :]
    wp = jnp.kron(eye, p['p_w'])
    bp = jnp.tile(p['p_b'], K)[None, :]
    g2t = jnp.tile(p['bn2_g'], K)[None, :]
    b2t = jnp.tile(p['bn2_b'], K)[None, :]

    return _head_tc(a4, sums2, g2t, b2t, wrf, brf, wbb, bbb, wp, bp)
